# baseline (speedup 1.0000x reference)
import functools

import jax
import jax.numpy as jnp
from jax import lax
from jax.experimental import pallas as pl
from jax.experimental.pallas import tpu as pltpu
from jax.experimental.pallas import tpu_sc as plsc

_B, _F, _T = 64, 128, 4096
_NS = 32
_NT = _B - _NS
_NW = 32
_BPW = _NS // _NW
_NTM = 10
_GR = 4
_NG = _F // _GR
_NBUF = 6
_LOOK = 2
_NCH = _T // 16

_mesh = plsc.VectorSubcoreMesh(core_axis_name="c", subcore_axis_name="s")


@functools.partial(
    pl.kernel,
    out_type=jax.ShapeDtypeStruct((_NS, _F, _T), jnp.float32),
    mesh=_mesh,
    scratch_types=[
        pltpu.VMEM((_NBUF, _GR, _T), jnp.float32),
        pltpu.VMEM((_T,), jnp.float32),
        pltpu.VMEM((_B, 16), jnp.int32),
        pltpu.VMEM((4, 16), jnp.int32),
        pltpu.SemaphoreType.DMA((_NBUF,)),
        pltpu.SemaphoreType.DMA((_NBUF,)),
    ],
)
def _sc_run(x_hbm, xlb_hbm, prm_hbm, out_hbm,
            gbuf, keeprow, xl_v, prm_v, sem_in, sem_out):
    wid = lax.axis_index("s") * 2 + lax.axis_index("c")

    zv = jnp.zeros((16,), jnp.float32)
    ones = jnp.ones((16,), jnp.float32)

    pltpu.sync_copy(xlb_hbm, xl_v)
    pltpu.sync_copy(prm_hbm, prm_v)

    ts_v = prm_v[0]
    tw_v = prm_v[1]
    fs_v = prm_v[2]
    fe_v = fs_v + prm_v[3]
    fs0, fe0 = fs_v[0], fe_v[0]
    fs1, fe1 = fs_v[1], fe_v[1]

    def _is_freq(f):
        return ((f >= fs0) & (f < fe0)) | ((f >= fs1) & (f < fe1))

    def _full_freq(g):
        full = _is_freq(g * _GR)
        for r in range(1, _GR):
            full = full & _is_freq(g * _GR + r)
        return full

    def _any_freq(g):
        anyf = _is_freq(g * _GR)
        for r in range(1, _GR):
            anyf = anyf | _is_freq(g * _GR + r)
        return anyf

    def _g_in(b, g, slot):
        off = pl.multiple_of(g * _GR, _GR)
        pltpu.async_copy(x_hbm.at[b, pl.ds(off, _GR), :], gbuf.at[slot],
                         sem_in.at[slot])

    def _g_in_wait(b, slot):
        pltpu.make_async_copy(x_hbm.at[b, pl.ds(0, _GR), :], gbuf.at[slot],
                              sem_in.at[slot]).wait()

    def _g_out(b, g, slot):
        off = pl.multiple_of(g * _GR, _GR)
        pltpu.async_copy(gbuf.at[slot], out_hbm.at[b, pl.ds(off, _GR), :],
                         sem_out.at[slot])

    def _g_out_wait(b, slot):
        pltpu.make_async_copy(gbuf.at[slot], out_hbm.at[b, pl.ds(0, _GR), :],
                              sem_out.at[slot]).wait()

    for bi in range(_BPW):
        b = wid * _BPW + bi
        xlv = xl_v[b]
        s_vec = jnp.minimum(ts_v, xlv)
        e_vec = jnp.minimum(ts_v + tw_v, xlv)
        c0_vec = (s_vec + 15) >> 4
        c1_vec = e_vec >> 4
        clo_vec = s_vec >> 4
        chi_vec = (e_vec + 15) >> 4

        def _init(i, carry):
            keeprow[pl.ds(i * 16, 16)] = ones
            return carry

        lax.fori_loop(0, _NCH, _init, 0)

        for i in range(_NTM):
            s_i, e_i = s_vec[i], e_vec[i]

            @pl.when(s_i < e_i)
            def _():
                def _zero(c, carry):
                    keeprow[pl.ds(c * 16, 16)] = zv
                    return carry

                lax.fori_loop(c0_vec[i], c1_vec[i], _zero, 0)

                def _edge(ec):
                    tvec = lax.iota(jnp.int32, 16) + ec * 16
                    m = (tvec >= s_i) & (tvec < e_i)
                    cur = keeprow[pl.ds(ec * 16, 16)]
                    keeprow[pl.ds(ec * 16, 16)] = jnp.where(m, 0.0, cur)

                fix_l = (s_i & 15) != 0
                fix_r = ((e_i & 15) != 0) & (
                    jnp.logical_not(fix_l) | ((e_i >> 4) != (s_i >> 4)))

                @pl.when(fix_l)
                def _():
                    _edge(s_i >> 4)

                @pl.when(fix_r)
                def _():
                    _edge(e_i >> 4)

        for g0 in range(_LOOK):
            @pl.when(jnp.logical_not(_full_freq(g0)))
            def _():
                _g_in(b, g0, g0 % _NBUF)

        def _gstep(g, carry):
            slot = g % _NBUF
            h = g + _LOOK

            @pl.when(h < _NG)
            def _():
                hslot = h % _NBUF

                @pl.when(h >= _NBUF)
                def _():
                    _g_out_wait(b, hslot)

                @pl.when(jnp.logical_not(_full_freq(h)))
                def _():
                    _g_in(b, h, hslot)

            full = _full_freq(g)

            @pl.when(full)
            def _():
                def _zg(c, carry2):
                    for r in range(_GR):
                        gbuf[slot, r, pl.ds(c * 16, 16)] = zv
                    return carry2

                lax.fori_loop(0, _NCH, _zg, 0)

            @pl.when(jnp.logical_not(full))
            def _():
                _g_in_wait(b, slot)

                @pl.when(_any_freq(g))
                def _():
                    for r in range(_GR):
                        @pl.when(_is_freq(g * _GR + r))
                        def _():
                            def _zr(c, carry2):
                                gbuf[slot, r, pl.ds(c * 16, 16)] = zv
                                return carry2

                            lax.fori_loop(0, _NCH, _zr, 0)

                for i in range(_NTM):
                    def _mul(c, carry2):
                        k = keeprow[pl.ds(c * 16, 16)]
                        for r in range(_GR):
                            v = gbuf[slot, r, pl.ds(c * 16, 16)]
                            gbuf[slot, r, pl.ds(c * 16, 16)] = v * k
                        return carry2

                    lax.fori_loop(clo_vec[i], chi_vec[i], _mul, 0)

            _g_out(b, g, slot)
            return carry

        lax.fori_loop(0, _NG, _gstep, 0)

        def _gdrain(g, carry):
            _g_out_wait(b, g % _NBUF)
            return carry

        lax.fori_loop(_NG - _NBUF, _NG, _gdrain, 0)


_BT = 2048


def _tc_body(xl_ref, fs_ref, fw_ref, ts_ref, tw_ref, x_ref, o_ref):
    b = pl.program_id(0) + _NS
    jt = pl.program_id(1)
    t0 = jt * _BT

    f_io = lax.broadcasted_iota(jnp.int32, (_F, 1), 0)
    fm = jnp.zeros((_F, 1), jnp.bool_)
    for i in range(fs_ref.shape[0]):
        s = fs_ref[i]
        fm = fm | ((f_io >= s) & (f_io < s + fw_ref[i]))
    a_f = jnp.where(fm, 0.0, 1.0).astype(jnp.float32)

    t_io = lax.broadcasted_iota(jnp.int32, (1, _BT), 1) + t0
    xl = xl_ref[b]
    tm = jnp.zeros((1, _BT), jnp.bool_)
    for i in range(ts_ref.shape[0]):
        s = ts_ref[i]
        tm = tm | ((t_io >= s) & (t_io < s + tw_ref[i]))
    tm = tm & (t_io < xl)
    a_t = jnp.where(tm, 0.0, 1.0).astype(jnp.float32)

    o_ref[0] = x_ref[0] * a_f * a_t


def _tc_run(x, xl, fs, fw, ts, tw):
    return pl.pallas_call(
        _tc_body,
        grid_spec=pltpu.PrefetchScalarGridSpec(
            num_scalar_prefetch=5,
            grid=(_NT, _T // _BT),
            in_specs=[
                pl.BlockSpec((1, _F, _BT), lambda b, jt, *_: (b + _NS, 0, jt)),
            ],
            out_specs=pl.BlockSpec((1, _F, _BT), lambda b, jt, *_: (b, 0, jt)),
        ),
        out_shape=jax.ShapeDtypeStruct((_NT, _F, _T), jnp.float32),
        compiler_params=pltpu.CompilerParams(
            dimension_semantics=("parallel", "parallel"),
        ),
    )(xl, fs, fw, ts, tw, x)


def kernel(x, x_len, freq_starts, freq_widths, time_starts, time_widths):
    xl = x_len.astype(jnp.int32)
    xlb = jnp.tile(xl[:, None], (1, 16))
    pad6 = jnp.zeros((6,), jnp.int32)
    pad14 = jnp.zeros((14,), jnp.int32)
    fs = freq_starts.astype(jnp.int32)
    fw = freq_widths.astype(jnp.int32)
    ts = time_starts.astype(jnp.int32)
    tw = time_widths.astype(jnp.int32)
    prm = jnp.stack([
        jnp.concatenate([ts, pad6]),
        jnp.concatenate([tw, pad6]),
        jnp.concatenate([fs, pad14]),
        jnp.concatenate([fw, pad14]),
    ])
    sc_out = _sc_run(x, xlb, prm)
    tc_out = _tc_run(x, xl, fs, fw, ts, tw)
    return jnp.concatenate([sc_out, tc_out], axis=0)


# speedup vs baseline: 1.8535x; 1.8535x over previous
import functools

import jax
import jax.numpy as jnp
from jax import lax
from jax.experimental import pallas as pl
from jax.experimental.pallas import tpu as pltpu
from jax.experimental.pallas import tpu_sc as plsc

_B, _F, _T = 64, 128, 4096
_NW = 32
_BPW = _B // _NW
_NTM = 10
_GR = 4
_NG = _F // _GR
_NBUF = 7
_LOOK = 3
_NCH = _T // 16

_mesh = plsc.VectorSubcoreMesh(core_axis_name="c", subcore_axis_name="s")


@functools.partial(
    pl.kernel,
    out_type=jax.ShapeDtypeStruct((_B, _F, _T), jnp.float32),
    mesh=_mesh,
    scratch_types=[
        pltpu.VMEM((_NBUF, _GR, _T), jnp.float32),
        pltpu.VMEM((_T,), jnp.float32),
        pltpu.VMEM((_B, 16), jnp.int32),
        pltpu.VMEM((4, 16), jnp.int32),
        pltpu.SemaphoreType.DMA((_NBUF,)),
        pltpu.SemaphoreType.DMA((_NBUF,)),
    ],
)
def _sc_run(x_hbm, xlb_hbm, prm_hbm, out_hbm,
            gbuf, keeprow, xl_v, prm_v, sem_in, sem_out):
    wid = lax.axis_index("s") * 2 + lax.axis_index("c")

    zv = jnp.zeros((16,), jnp.float32)
    ones = jnp.ones((16,), jnp.float32)

    pltpu.sync_copy(xlb_hbm, xl_v)
    pltpu.sync_copy(prm_hbm, prm_v)

    ts_v = prm_v[0]
    tw_v = prm_v[1]
    fs_v = prm_v[2]
    fe_v = fs_v + prm_v[3]
    fs0, fe0 = fs_v[0], fe_v[0]
    fs1, fe1 = fs_v[1], fe_v[1]

    def _is_freq(f):
        return ((f >= fs0) & (f < fe0)) | ((f >= fs1) & (f < fe1))

    def _full_freq(g):
        full = _is_freq(g * _GR)
        for r in range(1, _GR):
            full = full & _is_freq(g * _GR + r)
        return full

    def _any_freq(g):
        anyf = _is_freq(g * _GR)
        for r in range(1, _GR):
            anyf = anyf | _is_freq(g * _GR + r)
        return anyf

    def _g_in(b, g, slot):
        off = pl.multiple_of(g * _GR, _GR)
        pltpu.async_copy(x_hbm.at[b, pl.ds(off, _GR), :], gbuf.at[slot],
                         sem_in.at[slot])

    def _g_in_wait(b, slot):
        pltpu.make_async_copy(x_hbm.at[b, pl.ds(0, _GR), :], gbuf.at[slot],
                              sem_in.at[slot]).wait()

    def _g_out(b, g, slot):
        off = pl.multiple_of(g * _GR, _GR)
        pltpu.async_copy(gbuf.at[slot], out_hbm.at[b, pl.ds(off, _GR), :],
                         sem_out.at[slot])

    def _g_out_wait(b, slot):
        pltpu.make_async_copy(gbuf.at[slot], out_hbm.at[b, pl.ds(0, _GR), :],
                              sem_out.at[slot]).wait()

    for bi in range(_BPW):
        b = wid * _BPW + bi
        xlv = xl_v[b]
        s_vec = jnp.minimum(ts_v, xlv)
        e_vec = jnp.minimum(ts_v + tw_v, xlv)
        c0_vec = (s_vec + 15) >> 4
        c1_vec = e_vec >> 4
        clo_vec = s_vec >> 4
        chi_vec = (e_vec + 15) >> 4

        def _init(i, carry):
            keeprow[pl.ds(i * 16, 16)] = ones
            return carry

        lax.fori_loop(0, _NCH, _init, 0)

        for i in range(_NTM):
            s_i, e_i = s_vec[i], e_vec[i]

            @pl.when(s_i < e_i)
            def _():
                def _zero(c, carry):
                    keeprow[pl.ds(c * 16, 16)] = zv
                    return carry

                lax.fori_loop(c0_vec[i], c1_vec[i], _zero, 0)

                def _edge(ec):
                    tvec = lax.iota(jnp.int32, 16) + ec * 16
                    m = (tvec >= s_i) & (tvec < e_i)
                    cur = keeprow[pl.ds(ec * 16, 16)]
                    keeprow[pl.ds(ec * 16, 16)] = jnp.where(m, 0.0, cur)

                fix_l = (s_i & 15) != 0
                fix_r = ((e_i & 15) != 0) & (
                    jnp.logical_not(fix_l) | ((e_i >> 4) != (s_i >> 4)))

                @pl.when(fix_l)
                def _():
                    _edge(s_i >> 4)

                @pl.when(fix_r)
                def _():
                    _edge(e_i >> 4)

        for g0 in range(_LOOK):
            @pl.when(jnp.logical_not(_full_freq(g0)))
            def _():
                _g_in(b, g0, g0 % _NBUF)

        def _gstep(g, carry):
            slot = g % _NBUF
            h = g + _LOOK

            @pl.when(h < _NG)
            def _():
                hslot = h % _NBUF

                @pl.when(h >= _NBUF)
                def _():
                    _g_out_wait(b, hslot)

                @pl.when(jnp.logical_not(_full_freq(h)))
                def _():
                    _g_in(b, h, hslot)

            full = _full_freq(g)

            @pl.when(full)
            def _():
                def _zg(c, carry2):
                    for r in range(_GR):
                        gbuf[slot, r, pl.ds(c * 16, 16)] = zv
                    return carry2

                lax.fori_loop(0, _NCH, _zg, 0)

            @pl.when(jnp.logical_not(full))
            def _():
                _g_in_wait(b, slot)

                @pl.when(_any_freq(g))
                def _():
                    for r in range(_GR):
                        @pl.when(_is_freq(g * _GR + r))
                        def _():
                            def _zr(c, carry2):
                                gbuf[slot, r, pl.ds(c * 16, 16)] = zv
                                return carry2

                            lax.fori_loop(0, _NCH, _zr, 0)

                for i in range(_NTM):
                    def _mul(c, carry2):
                        k = keeprow[pl.ds(c * 16, 16)]
                        for r in range(_GR):
                            v = gbuf[slot, r, pl.ds(c * 16, 16)]
                            gbuf[slot, r, pl.ds(c * 16, 16)] = v * k
                        return carry2

                    lax.fori_loop(clo_vec[i], chi_vec[i], _mul, 0)

            _g_out(b, g, slot)
            return carry

        lax.fori_loop(0, _NG, _gstep, 0)

        def _gdrain(g, carry):
            _g_out_wait(b, g % _NBUF)
            return carry

        lax.fori_loop(_NG - _NBUF, _NG, _gdrain, 0)


def kernel(x, x_len, freq_starts, freq_widths, time_starts, time_widths):
    xl = x_len.astype(jnp.int32)
    xlb = jnp.tile(xl[:, None], (1, 16))
    pad6 = jnp.zeros((6,), jnp.int32)
    pad14 = jnp.zeros((14,), jnp.int32)
    prm = jnp.stack([
        jnp.concatenate([time_starts.astype(jnp.int32), pad6]),
        jnp.concatenate([time_widths.astype(jnp.int32), pad6]),
        jnp.concatenate([freq_starts.astype(jnp.int32), pad14]),
        jnp.concatenate([freq_widths.astype(jnp.int32), pad14]),
    ])
    return _sc_run(x, xlb, prm)
